# baseline (speedup 1.0000x reference)
import functools

import jax
import jax.numpy as jnp
from jax import lax
from jax.experimental import pallas as pl
from jax.experimental.pallas import tpu as pltpu
from jax.experimental.pallas import tpu_sc as plsc

N_VOCAB = 123
HIDDEN = 1000
N_OUT = 123
B = 4096
L = 20
_LP = 24
_NSPLIT = 2
_BH = B // _NSPLIT

_NC = 2
_NS = 16
_NW = _NC * _NS
_CH = 128
_NCHUNK = _BH * _LP // (_NW * _CH)
_SUP = 2
_NSUP = _NCHUNK // _SUP
_NBUF = 3
_DPAD = 128

_SLICE_BB = 128


def _table_body(emb_ref, w1_ref, b1_ref, w2_ref, b2_ref, out_ref):
    h = jnp.maximum(emb_ref[...], 0.0)
    h = jnp.dot(h, w1_ref[...], preferred_element_type=jnp.float32) + b1_ref[...]
    h = jnp.maximum(h, 0.0)
    t = jnp.dot(h, w2_ref[...], preferred_element_type=jnp.float32) + b2_ref[...]
    out_ref[...] = jnp.pad(
        t, ((0, _DPAD - N_VOCAB), (0, _DPAD - N_OUT))
    )


_table_call = pl.pallas_call(
    _table_body,
    out_shape=jax.ShapeDtypeStruct((_DPAD, _DPAD), jnp.float32),
)


@functools.cache
def _make_gather_call():
    mesh = plsc.VectorSubcoreMesh(core_axis_name="c", subcore_axis_name="s")

    @functools.partial(
        pl.kernel,
        mesh=mesh,
        out_type=jax.ShapeDtypeStruct((_BH, _LP, _DPAD), jnp.float32),
        scratch_types=[
            pltpu.VMEM((_NCHUNK, _CH), jnp.int32),
            pltpu.VMEM((_NBUF, _SUP * _CH, _DPAD), jnp.float32),
            pltpu.VMEM_SHARED((_DPAD, _DPAD), jnp.float32),
            pltpu.SemaphoreType.DMA,
            pltpu.SemaphoreType.DMA,
            pltpu.SemaphoreType.DMA,
            pltpu.SemaphoreType.DMA,
        ],
    )
    def _gather_call(
        idx_hbm, table_hbm, out_hbm, idx_v, rows, table_sp, g0, g1, g2, wsem
    ):
        sid = lax.axis_index("s")
        wid = sid * _NC + lax.axis_index("c")
        @pl.when(sid == 0)
        def _():
            pltpu.sync_copy(table_hbm, table_sp)

        pltpu.sync_copy(idx_hbm.at[wid], idx_v)
        plsc.subcore_barrier()
        out_flat = out_hbm.reshape(_BH * _LP, _DPAD)
        gsems = (g0, g1, g2)
        wcopies = [None] * _NSUP
        for s in range(_NSUP):
            buf = s % _NBUF
            if s >= _NBUF:
                wcopies[s - _NBUF].wait()
            gcopies = [
                pltpu.async_copy(
                    table_sp.at[idx_v.at[s * _SUP + k]],
                    rows.at[buf, pl.ds(k * _CH, _CH)],
                    gsems[buf],
                )
                for k in range(_SUP)
            ]
            for cp in gcopies:
                cp.wait()
            wcopies[s] = pltpu.async_copy(
                rows.at[buf],
                out_flat.at[
                    pl.ds(wid * _NCHUNK * _CH + s * _SUP * _CH, _SUP * _CH)
                ],
                wsem,
            )
        for s in range(_NSUP - _NBUF, _NSUP):
            wcopies[s].wait()

    return _gather_call


def _slice_body0(h_ref, out_ref):
    out_ref[...] = h_ref[:, :L, :N_OUT]


def _slice_body1(h_ref, acc_ref, out_ref):
    del acc_ref
    out_ref[...] = h_ref[:, :L, :N_OUT]


_NBLK = _BH // _SLICE_BB


def _h_map(i):
    return (i, 0, 0)


_slice_call0 = pl.pallas_call(
    _slice_body0,
    grid=(_NBLK,),
    in_specs=[pl.BlockSpec((_SLICE_BB, _LP, _DPAD), _h_map)],
    out_specs=pl.BlockSpec((_SLICE_BB, L, N_OUT), _h_map),
    out_shape=jax.ShapeDtypeStruct((B, L, N_OUT), jnp.float32),
)

_slice_call1 = pl.pallas_call(
    _slice_body1,
    grid=(_NBLK,),
    in_specs=[
        pl.BlockSpec((_SLICE_BB, _LP, _DPAD), _h_map),
        pl.BlockSpec(memory_space=pl.ANY),
    ],
    out_specs=pl.BlockSpec((_SLICE_BB, L, N_OUT), lambda i: (_NBLK + i, 0, 0)),
    out_shape=jax.ShapeDtypeStruct((B, L, N_OUT), jnp.float32),
    input_output_aliases={1: 0},
)


def kernel(inputs, embed, W1, b1, W2, b2):
    table = _table_call(
        embed, W1, b1.reshape(1, HIDDEN), W2, b2.reshape(1, N_OUT)
    )
    idx = jnp.pad(inputs.astype(jnp.int32), ((0, 0), (0, _LP - L)))
    idx = idx.reshape(_NSPLIT, _NW, _NCHUNK, _CH)
    gather = _make_gather_call()
    h0 = gather(idx[0], table)
    h1 = gather(idx[1], table)
    acc = _slice_call0(h0)
    return _slice_call1(h1, acc)


# speedup vs baseline: 1.3831x; 1.3831x over previous
import functools

import jax
import jax.numpy as jnp
from jax import lax
from jax.experimental import pallas as pl
from jax.experimental.pallas import tpu as pltpu
from jax.experimental.pallas import tpu_sc as plsc

N_VOCAB = 123
HIDDEN = 1000
N_OUT = 123
B = 4096
L = 20
_LP = 24

_NC = 2
_NS = 16
_NW = _NC * _NS
_CH = 128
_NCHUNK = B * _LP // (_NW * _CH)
_SUP = 3
_NSUP = _NCHUNK // _SUP
_NBUF = 2
_DPAD = 128


def _table_body(emb_ref, w1_ref, b1_ref, w2_ref, b2_ref, out_ref):
    h = jnp.maximum(emb_ref[...], 0.0)
    h = jnp.dot(h, w1_ref[...], preferred_element_type=jnp.float32) + b1_ref[...]
    h = jnp.maximum(h, 0.0)
    t = jnp.dot(h, w2_ref[...], preferred_element_type=jnp.float32) + b2_ref[...]
    out_ref[...] = jnp.pad(
        t, ((0, _DPAD - N_VOCAB), (0, _DPAD - N_OUT))
    )


_table_call = pl.pallas_call(
    _table_body,
    out_shape=jax.ShapeDtypeStruct((_DPAD, _DPAD), jnp.float32),
)


@functools.cache
def _make_gather_call():
    mesh = plsc.VectorSubcoreMesh(core_axis_name="c", subcore_axis_name="s")

    @functools.partial(
        pl.kernel,
        mesh=mesh,
        out_type=jax.ShapeDtypeStruct((B, _LP, _DPAD), jnp.float32),
        scratch_types=[
            pltpu.VMEM((_NCHUNK, _CH), jnp.int32),
            pltpu.VMEM((_NBUF, _SUP * _CH, _DPAD), jnp.float32),
            pltpu.VMEM_SHARED((_DPAD, _DPAD), jnp.float32),
            pltpu.SemaphoreType.DMA,
            pltpu.SemaphoreType.DMA,
            pltpu.SemaphoreType.DMA,
        ],
    )
    def _gather_call(
        idx_hbm, table_hbm, out_hbm, idx_v, rows, table_sp, g0, g1, wsem
    ):
        sid = lax.axis_index("s")
        wid = sid * _NC + lax.axis_index("c")
        @pl.when(sid == 0)
        def _():
            pltpu.sync_copy(table_hbm, table_sp)

        pltpu.sync_copy(idx_hbm.at[wid], idx_v)
        plsc.subcore_barrier()
        out_flat = out_hbm.reshape(B * _LP, _DPAD)
        gsems = (g0, g1)
        wcopies = [None] * _NSUP
        for s in range(_NSUP):
            buf = s % _NBUF
            if s >= _NBUF:
                wcopies[s - _NBUF].wait()
            gcopies = [
                pltpu.async_copy(
                    table_sp.at[idx_v.at[s * _SUP + k]],
                    rows.at[buf, pl.ds(k * _CH, _CH)],
                    gsems[buf],
                )
                for k in range(_SUP)
            ]
            for cp in gcopies:
                cp.wait()
            wcopies[s] = pltpu.async_copy(
                rows.at[buf],
                out_flat.at[
                    pl.ds(wid * _NCHUNK * _CH + s * _SUP * _CH, _SUP * _CH)
                ],
                wsem,
            )
        for s in range(_NSUP - _NBUF, _NSUP):
            wcopies[s].wait()

    return _gather_call


def kernel(inputs, embed, W1, b1, W2, b2):
    table = _table_call(
        embed, W1, b1.reshape(1, HIDDEN), W2, b2.reshape(1, N_OUT)
    )
    idx = jnp.pad(inputs.astype(jnp.int32), ((0, 0), (0, _LP - L)))
    idx = idx.reshape(_NW, _NCHUNK, _CH)
    out = _make_gather_call()(idx, table)
    return out[:, :L, :N_OUT]


# speedup vs baseline: 1.3853x; 1.0016x over previous
import functools

import jax
import jax.numpy as jnp
from jax import lax
from jax.experimental import pallas as pl
from jax.experimental.pallas import tpu as pltpu
from jax.experimental.pallas import tpu_sc as plsc

N_VOCAB = 123
HIDDEN = 1000
N_OUT = 123
B = 4096
L = 20
_LP = 24

_NC = 2
_NS = 16
_NW = _NC * _NS
_CH = 128
_NCHUNK = B * _LP // (_NW * _CH)
_SUP = 3
_NSUP = _NCHUNK // _SUP
_NBUF = 2
_DPAD = 128


def _table_body(emb_ref, w1_ref, b1_ref, w2_ref, b2_ref, out_ref):
    h = jnp.maximum(emb_ref[...], 0.0)
    h = jnp.dot(h, w1_ref[...], preferred_element_type=jnp.float32)
    h = jnp.maximum(h + b1_ref[...][None, :], 0.0)
    t = jnp.dot(h, w2_ref[...], preferred_element_type=jnp.float32)
    t = t + b2_ref[...][None, :]
    out_ref[...] = jnp.pad(
        t, ((0, _DPAD - N_VOCAB), (0, _DPAD - N_OUT))
    )


_table_call = pl.pallas_call(
    _table_body,
    out_shape=jax.ShapeDtypeStruct((_DPAD, _DPAD), jnp.float32),
)


@functools.cache
def _make_gather_call():
    mesh = plsc.VectorSubcoreMesh(core_axis_name="c", subcore_axis_name="s")

    @functools.partial(
        pl.kernel,
        mesh=mesh,
        out_type=jax.ShapeDtypeStruct((B, _LP, _DPAD), jnp.float32),
        scratch_types=[
            pltpu.VMEM((_NCHUNK, _CH), jnp.int32),
            pltpu.VMEM((_NBUF, _SUP * _CH, _DPAD), jnp.float32),
            pltpu.VMEM_SHARED((_DPAD, _DPAD), jnp.float32),
            pltpu.SemaphoreType.DMA,
            pltpu.SemaphoreType.DMA,
            pltpu.SemaphoreType.DMA,
        ],
    )
    def _gather_call(
        idx_hbm, table_hbm, out_hbm, idx_v, rows, table_sp, g0, g1, wsem
    ):
        sid = lax.axis_index("s")
        wid = sid * _NC + lax.axis_index("c")
        @pl.when(sid == 0)
        def _():
            pltpu.sync_copy(table_hbm, table_sp)

        pltpu.sync_copy(idx_hbm.at[wid], idx_v)
        plsc.subcore_barrier()
        out_flat = out_hbm.reshape(B * _LP, _DPAD)
        gsems = (g0, g1)
        wcopies = [None] * _NSUP
        for s in range(_NSUP):
            buf = s % _NBUF
            if s >= _NBUF:
                wcopies[s - _NBUF].wait()
            gcopies = [
                pltpu.async_copy(
                    table_sp.at[idx_v.at[s * _SUP + k]],
                    rows.at[buf, pl.ds(k * _CH, _CH)],
                    gsems[buf],
                )
                for k in range(_SUP)
            ]
            for cp in gcopies:
                cp.wait()
            wcopies[s] = pltpu.async_copy(
                rows.at[buf],
                out_flat.at[
                    pl.ds(wid * _NCHUNK * _CH + s * _SUP * _CH, _SUP * _CH)
                ],
                wsem,
            )
        for s in range(_NSUP - _NBUF, _NSUP):
            wcopies[s].wait()

    return _gather_call


def kernel(inputs, embed, W1, b1, W2, b2):
    table = _table_call(embed, W1, b1, W2, b2)
    idx = jnp.pad(inputs.astype(jnp.int32), ((0, 0), (0, _LP - L)))
    idx = idx.reshape(_NW, _NCHUNK, _CH)
    out = _make_gather_call()(idx, table)
    return out[:, :L, :N_OUT]
